# baseline (speedup 1.0000x reference)
import functools

import numpy as np
import jax
import jax.numpy as jnp
from jax import lax
from jax.experimental import pallas as pl
from jax.experimental.pallas import tpu as pltpu
from jax.experimental.pallas import tpu_sc as plsc

_DIM = 64
_NB = 4096
_NS = 200
_NW = 32
_BT = _NB // 128
_HALF = 500_224
_BANDWIDTH = 2.0


def _gaussian_kernel_const() -> np.ndarray:
    t = np.arange(24, dtype=np.float32)
    ad = np.abs(t[None, :] - t[:, None])
    dist = np.minimum(ad, 24.0 - ad)
    return np.exp(-0.5 * (dist / _BANDWIDTH) ** 2).astype(np.float32)


_KMAT = _gaussian_kernel_const()


def _tc_pack(table_t):
    w = 512
    g = _HALF // w

    def body(a_ref, b_ref, eye_ref, out_ref):
        eye = eye_ref[...]
        dn = (((0,), (0,)), ((), ()))
        out_ref[:, 0:_DIM] = lax.dot_general(
            a_ref[...], eye, dn, preferred_element_type=jnp.float32)
        out_ref[:, _DIM:128] = lax.dot_general(
            b_ref[...], eye, dn, preferred_element_type=jnp.float32)

    return pl.pallas_call(
        body,
        grid=(g,),
        in_specs=[
            pl.BlockSpec((_DIM, w), lambda i: (0, i)),
            pl.BlockSpec((_DIM, w), lambda i: (0, i + g)),
            pl.BlockSpec((_DIM, _DIM), lambda i: (0, 0)),
        ],
        out_specs=pl.BlockSpec((w, 128), lambda i: (i, 0)),
        out_shape=jax.ShapeDtypeStruct((_HALF, 128), jnp.float32),
    )(table_t, table_t, jnp.eye(_DIM, dtype=jnp.float32))


def _sc_gather_fused(idx3, table2):
    mesh = plsc.VectorSubcoreMesh(core_axis_name="c", subcore_axis_name="s")

    @functools.partial(
        pl.kernel,
        mesh=mesh,
        out_type=jax.ShapeDtypeStruct((_NS, 8, _BT, 8, 128), jnp.float32),
        scratch_types=[
            pltpu.VMEM((_NS, 128), jnp.int32),
            pltpu.VMEM((2, 128, _DIM), jnp.float32),
            pltpu.VMEM((2, 8, 8, 128), jnp.float32),
            pltpu.SemaphoreType.DMA,
            pltpu.SemaphoreType.DMA,
            pltpu.SemaphoreType.DMA,
            pltpu.SemaphoreType.DMA,
        ],
        compiler_params=pltpu.CompilerParams(
            use_tc_tiling_on_sc=False, needs_layout_passes=False
        ),
    )
    def gather_kernel(idx_hbm, table_hbm, out_hbm,
                      idxv, rowbuf, tbuf, gsem0, gsem1, wsem0, wsem1):
        w = lax.axis_index("s") * 2 + lax.axis_index("c")
        gsems = (gsem0, gsem1)
        wsems = (wsem0, wsem1)
        pltpu.sync_copy(idx_hbm.at[:, w], idxv)

        lvec = lax.iota(jnp.int32, 16)
        doffs = [(lvec + i) & 15 for i in range(16)]

        def gather_desc(s, b):
            return pltpu.make_async_copy(
                table_hbm.at[idxv.at[s]], rowbuf.at[b], gsems[b])

        def write_desc(s, b):
            return pltpu.make_async_copy(
                tbuf.at[b], out_hbm.at[s, :, w], wsems[b])

        def transpose(b):
            def blk(kb, carry):
                rows = kb * 16 + lvec
                for kd in range(4):
                    for step in range(16):
                        doff = doffs[step]
                        v = plsc.load_gather(
                            rowbuf.at[b], [rows, kd * 16 + doff])
                        plsc.store_scatter(
                            tbuf.at[b],
                            [2 * kd + (doff >> 3), doff & 7, rows], v)
                return carry

            lax.fori_loop(0, 8, blk, 0)

        gather_desc(0, 0).start()
        gather_desc(1, 1).start()

        def body(g, carry):
            for b in (0, 1):
                s = 2 * g + b
                gather_desc(s, b).wait()

                @pl.when(g >= 1)
                def _drain():
                    write_desc(s - 2, b).wait()

                transpose(b)

                @pl.when(g < _NS // 2 - 1)
                def _next():
                    gather_desc(s + 2, b).start()

                write_desc(s, b).start()

            return carry

        lax.fori_loop(0, _NS // 2, body, 0)
        write_desc(_NS - 2, 0).wait()
        write_desc(_NS - 1, 1).wait()

    return gather_kernel(idx3, table2.reshape(2 * _HALF, _DIM))


def _tc_small(user_t, time_table, kmat):
    nu = user_t.shape[1]
    blk = 8192
    grid = (nu + blk - 1) // blk

    def body(user_ref, time_ref, kmat_ref, user_out, time_out, smooth_out):
        i = pl.program_id(0)
        user_out[...] = user_ref[...]

        @pl.when(i == 0)
        def _():
            time_out[...] = time_ref[...]
            smooth_out[...] = jnp.dot(
                kmat_ref[...], time_ref[...], preferred_element_type=jnp.float32
            )

    return pl.pallas_call(
        body,
        grid=(grid,),
        in_specs=[
            pl.BlockSpec((_DIM, blk), lambda i: (0, i)),
            pl.BlockSpec((24, _DIM), lambda i: (0, 0)),
            pl.BlockSpec((24, 24), lambda i: (0, 0)),
        ],
        out_specs=[
            pl.BlockSpec((_DIM, blk), lambda i: (0, i)),
            pl.BlockSpec((24, _DIM), lambda i: (0, 0)),
            pl.BlockSpec((24, _DIM), lambda i: (0, 0)),
        ],
        out_shape=[
            jax.ShapeDtypeStruct((_DIM, nu), jnp.float32),
            jax.ShapeDtypeStruct((24, _DIM), jnp.float32),
            jax.ShapeDtypeStruct((24, _DIM), jnp.float32),
        ],
    )(user_t, time_table, kmat)


def kernel(location_x, loc_table, user_table, time_table):
    locs = jnp.transpose(location_x.astype(jnp.int32)).reshape(_NS, _BT, 128)
    hi = locs >= _HALF
    idx3 = jnp.where(hi, 2 * (locs - _HALF) + 1, 2 * locs)
    table2 = _tc_pack(loc_table.T)
    x = _sc_gather_fused(idx3, table2)
    loc_embedded = x.transpose(2, 4, 0, 1, 3).reshape(_NB, _NS, _DIM)
    user_t_out, time_out, smooth_out = _tc_small(
        user_table.T, time_table, jnp.asarray(_KMAT)
    )
    return (loc_embedded, time_out, smooth_out, user_t_out.T)


# speedup vs baseline: 1.7369x; 1.7369x over previous
import functools

import numpy as np
import jax
import jax.numpy as jnp
from jax import lax
from jax.experimental import pallas as pl
from jax.experimental.pallas import tpu as pltpu
from jax.experimental.pallas import tpu_sc as plsc

_DIM = 64
_NB = 4096
_NS = 200
_NW = 32
_BT = _NB // 128
_HALF = 524_288
_BANDWIDTH = 2.0


def _gaussian_kernel_const() -> np.ndarray:
    t = np.arange(24, dtype=np.float32)
    ad = np.abs(t[None, :] - t[:, None])
    dist = np.minimum(ad, 24.0 - ad)
    return np.exp(-0.5 * (dist / _BANDWIDTH) ** 2).astype(np.float32)


_KMAT = _gaussian_kernel_const()


def _tc_pack(table_t):
    w = 2048
    g = _HALF // w
    jmax = 1_000_000 // w

    def body(a_ref, b_ref, eye_ref, out_ref):
        eye = eye_ref[...]
        dn = (((0,), (0,)), ((), ()))
        out_ref[:, 0:_DIM] = lax.dot_general(
            a_ref[...], eye, dn, preferred_element_type=jnp.float32)
        out_ref[:, _DIM:128] = lax.dot_general(
            b_ref[...], eye, dn, preferred_element_type=jnp.float32)

    return pl.pallas_call(
        body,
        grid=(g,),
        in_specs=[
            pl.BlockSpec((_DIM, w), lambda i: (0, i)),
            pl.BlockSpec((_DIM, w), lambda i: (0, jnp.minimum(i + g, jmax))),
            pl.BlockSpec((_DIM, _DIM), lambda i: (0, 0)),
        ],
        out_specs=pl.BlockSpec((w, 128), lambda i: (i, 0)),
        out_shape=jax.ShapeDtypeStruct((_HALF, 128), jnp.float32),
    )(table_t, table_t, jnp.eye(_DIM, dtype=jnp.float32))


def _sc_gather_fused(idx3, par3, table2):
    mesh = plsc.VectorSubcoreMesh(core_axis_name="c", subcore_axis_name="s")

    @functools.partial(
        pl.kernel,
        mesh=mesh,
        out_type=jax.ShapeDtypeStruct((_NS, 8, _BT, 8, 128), jnp.float32),
        scratch_types=[
            pltpu.VMEM((_NS, 128), jnp.int32),
            pltpu.VMEM((_NS, 128), jnp.int32),
            pltpu.VMEM((2, 128, 128), jnp.float32),
            pltpu.VMEM((2, 8, 8, 128), jnp.float32),
            pltpu.SemaphoreType.DMA,
            pltpu.SemaphoreType.DMA,
            pltpu.SemaphoreType.DMA,
            pltpu.SemaphoreType.DMA,
        ],
        compiler_params=pltpu.CompilerParams(
            use_tc_tiling_on_sc=False, needs_layout_passes=False
        ),
    )
    def gather_kernel(idx_hbm, par_hbm, table_hbm, out_hbm,
                      idxv, parv, rowbuf, tbuf, gsem0, gsem1, wsem0, wsem1):
        w = lax.axis_index("s") * 2 + lax.axis_index("c")
        gsems = (gsem0, gsem1)
        wsems = (wsem0, wsem1)
        pltpu.sync_copy(idx_hbm.at[:, w], idxv)
        pltpu.sync_copy(par_hbm.at[:, w], parv)

        lvec = lax.iota(jnp.int32, 16)
        doffs = [(lvec + i) & 15 for i in range(16)]

        def gather_desc(s, b):
            return pltpu.make_async_copy(
                table_hbm.at[idxv.at[s]], rowbuf.at[b], gsems[b])

        def write_desc(s, b):
            return pltpu.make_async_copy(
                tbuf.at[b], out_hbm.at[s, :, w], wsems[b])

        def transpose(s, b):
            def blk(kb, carry):
                rows = kb * 16 + lvec
                par16 = parv[s, pl.ds(kb * 16, 16)]
                for kd in range(4):
                    base = kd * 16 + par16
                    for step in range(16):
                        doff = doffs[step]
                        v = plsc.load_gather(
                            rowbuf.at[b], [rows, base + doff])
                        plsc.store_scatter(
                            tbuf.at[b],
                            [2 * kd + (doff >> 3), doff & 7, rows], v)
                return carry

            lax.fori_loop(0, 8, blk, 0)

        gather_desc(0, 0).start()
        gather_desc(1, 1).start()

        def body(g, carry):
            for b in (0, 1):
                s = 2 * g + b
                gather_desc(s, b).wait()

                @pl.when(g >= 1)
                def _drain():
                    write_desc(s - 2, b).wait()

                transpose(s, b)

                @pl.when(g < _NS // 2 - 1)
                def _next():
                    gather_desc(s + 2, b).start()

                write_desc(s, b).start()

            return carry

        lax.fori_loop(0, _NS // 2, body, 0)
        write_desc(_NS - 2, 0).wait()
        write_desc(_NS - 1, 1).wait()

    return gather_kernel(idx3, par3, table2)


def _tc_small(user_t, time_table, kmat):
    nu = user_t.shape[1]
    blk = 8192
    grid = (nu + blk - 1) // blk

    def body(user_ref, time_ref, kmat_ref, user_out, time_out, smooth_out):
        i = pl.program_id(0)
        user_out[...] = user_ref[...]

        @pl.when(i == 0)
        def _():
            time_out[...] = time_ref[...]
            smooth_out[...] = jnp.dot(
                kmat_ref[...], time_ref[...], preferred_element_type=jnp.float32
            )

    return pl.pallas_call(
        body,
        grid=(grid,),
        in_specs=[
            pl.BlockSpec((_DIM, blk), lambda i: (0, i)),
            pl.BlockSpec((24, _DIM), lambda i: (0, 0)),
            pl.BlockSpec((24, 24), lambda i: (0, 0)),
        ],
        out_specs=[
            pl.BlockSpec((_DIM, blk), lambda i: (0, i)),
            pl.BlockSpec((24, _DIM), lambda i: (0, 0)),
            pl.BlockSpec((24, _DIM), lambda i: (0, 0)),
        ],
        out_shape=[
            jax.ShapeDtypeStruct((_DIM, nu), jnp.float32),
            jax.ShapeDtypeStruct((24, _DIM), jnp.float32),
            jax.ShapeDtypeStruct((24, _DIM), jnp.float32),
        ],
    )(user_t, time_table, kmat)


def kernel(location_x, loc_table, user_table, time_table):
    locs = jnp.transpose(location_x.astype(jnp.int32)).reshape(_NS, _BT, 128)
    hi = locs >= _HALF
    idx3 = jnp.where(hi, locs - _HALF, locs)
    par3 = jnp.where(hi, _DIM, 0).astype(jnp.int32)
    table2 = _tc_pack(loc_table.T)
    x = _sc_gather_fused(idx3, par3, table2)
    loc_embedded = x.transpose(2, 4, 0, 1, 3).reshape(_NB, _NS, _DIM)
    user_t_out, time_out, smooth_out = _tc_small(
        user_table.T, time_table, jnp.asarray(_KMAT)
    )
    return (loc_embedded, time_out, smooth_out, user_t_out.T)


# speedup vs baseline: 1.9097x; 1.0995x over previous
import functools

import numpy as np
import jax
import jax.numpy as jnp
from jax import lax
from jax.experimental import pallas as pl
from jax.experimental.pallas import tpu as pltpu
from jax.experimental.pallas import tpu_sc as plsc

_DIM = 64
_NB = 4096
_NS = 200
_NW = 32
_BT = _NB // 128
_HALF = 524_288
_BANDWIDTH = 2.0


def _gaussian_kernel_const() -> np.ndarray:
    t = np.arange(24, dtype=np.float32)
    ad = np.abs(t[None, :] - t[:, None])
    dist = np.minimum(ad, 24.0 - ad)
    return np.exp(-0.5 * (dist / _BANDWIDTH) ** 2).astype(np.float32)


_KMAT = _gaussian_kernel_const()


def _tc_pack(table_t):
    w = 4096
    g = _HALF // w
    jmax = 1_000_000 // w

    def body(a_ref, b_ref, eye_ref, out_ref):
        eye = eye_ref[...]
        dn = (((0,), (0,)), ((), ()))
        out_ref[:, 0:_DIM] = lax.dot_general(
            a_ref[...], eye, dn, preferred_element_type=jnp.float32)
        out_ref[:, _DIM:128] = lax.dot_general(
            b_ref[...], eye, dn, preferred_element_type=jnp.float32)

    return pl.pallas_call(
        body,
        grid=(g,),
        in_specs=[
            pl.BlockSpec((_DIM, w), lambda i: (0, i)),
            pl.BlockSpec((_DIM, w), lambda i: (0, jnp.minimum(i + g, jmax))),
            pl.BlockSpec((_DIM, _DIM), lambda i: (0, 0)),
        ],
        out_specs=pl.BlockSpec((w, 128), lambda i: (i, 0)),
        out_shape=jax.ShapeDtypeStruct((_HALF, 128), jnp.float32),
    )(table_t, table_t, jnp.eye(_DIM, dtype=jnp.float32))


def _sc_gather_fused(idx3, par3, table2):
    mesh = plsc.VectorSubcoreMesh(core_axis_name="c", subcore_axis_name="s")

    @functools.partial(
        pl.kernel,
        mesh=mesh,
        out_type=jax.ShapeDtypeStruct((_NS, 8, _BT, 8, 128), jnp.float32),
        scratch_types=[
            pltpu.VMEM((_NS, 128), jnp.int32),
            pltpu.VMEM((_NS, 128), jnp.int32),
            pltpu.VMEM((2, 128, 128), jnp.float32),
            pltpu.VMEM((2, 8, 8, 128), jnp.float32),
            pltpu.SemaphoreType.DMA,
            pltpu.SemaphoreType.DMA,
            pltpu.SemaphoreType.DMA,
            pltpu.SemaphoreType.DMA,
        ],
        compiler_params=pltpu.CompilerParams(
            use_tc_tiling_on_sc=False, needs_layout_passes=False
        ),
    )
    def gather_kernel(idx_hbm, par_hbm, table_hbm, out_hbm,
                      idxv, parv, rowbuf, tbuf, gsem0, gsem1, wsem0, wsem1):
        w = lax.axis_index("s") * 2 + lax.axis_index("c")
        gsems = (gsem0, gsem1)
        wsems = (wsem0, wsem1)
        pltpu.sync_copy(idx_hbm.at[:, w], idxv)
        pltpu.sync_copy(par_hbm.at[:, w], parv)

        lvec = lax.iota(jnp.int32, 16)
        doffs = [(lvec + i) & 15 for i in range(16)]

        def gather_desc(s, b):
            return pltpu.make_async_copy(
                table_hbm.at[idxv.at[s]], rowbuf.at[b], gsems[b])

        def write_desc(s, b):
            return pltpu.make_async_copy(
                tbuf.at[b], out_hbm.at[s, :, w], wsems[b])

        def transpose(s, b):
            def blk(kb, carry):
                rows = kb * 16 + lvec
                par16 = parv[s, pl.ds(kb * 16, 16)]
                for kd in range(4):
                    base = kd * 16 + par16
                    for step in range(16):
                        doff = doffs[step]
                        v = plsc.load_gather(
                            rowbuf.at[b], [rows, base + doff])
                        plsc.store_scatter(
                            tbuf.at[b],
                            [2 * kd + (doff >> 3), doff & 7, rows], v)
                return carry

            lax.fori_loop(0, 8, blk, 0)

        gather_desc(0, 0).start()
        gather_desc(1, 1).start()

        def body(g, carry):
            for b in (0, 1):
                s = 2 * g + b
                gather_desc(s, b).wait()

                @pl.when(g >= 1)
                def _drain():
                    write_desc(s - 2, b).wait()

                transpose(s, b)

                @pl.when(g < _NS // 2 - 1)
                def _next():
                    gather_desc(s + 2, b).start()

                write_desc(s, b).start()

            return carry

        lax.fori_loop(0, _NS // 2, body, 0)
        write_desc(_NS - 2, 0).wait()
        write_desc(_NS - 1, 1).wait()

    return gather_kernel(idx3, par3, table2)


def _tc_small(user_t, time_table, kmat):
    nu = user_t.shape[1]
    blk = 8192
    grid = (nu + blk - 1) // blk

    def body(user_ref, time_ref, kmat_ref, user_out, time_out, smooth_out):
        i = pl.program_id(0)
        user_out[...] = user_ref[...]

        @pl.when(i == 0)
        def _():
            time_out[...] = time_ref[...]
            smooth_out[...] = jnp.dot(
                kmat_ref[...], time_ref[...], preferred_element_type=jnp.float32
            )

    return pl.pallas_call(
        body,
        grid=(grid,),
        in_specs=[
            pl.BlockSpec((_DIM, blk), lambda i: (0, i)),
            pl.BlockSpec((24, _DIM), lambda i: (0, 0)),
            pl.BlockSpec((24, 24), lambda i: (0, 0)),
        ],
        out_specs=[
            pl.BlockSpec((_DIM, blk), lambda i: (0, i)),
            pl.BlockSpec((24, _DIM), lambda i: (0, 0)),
            pl.BlockSpec((24, _DIM), lambda i: (0, 0)),
        ],
        out_shape=[
            jax.ShapeDtypeStruct((_DIM, nu), jnp.float32),
            jax.ShapeDtypeStruct((24, _DIM), jnp.float32),
            jax.ShapeDtypeStruct((24, _DIM), jnp.float32),
        ],
    )(user_t, time_table, kmat)


def kernel(location_x, loc_table, user_table, time_table):
    locs = jnp.transpose(location_x.astype(jnp.int32)).reshape(_NS, _BT, 128)
    hi = locs >= _HALF
    idx3 = jnp.where(hi, locs - _HALF, locs)
    par3 = jnp.where(hi, _DIM, 0).astype(jnp.int32)
    table2 = _tc_pack(loc_table.T)
    x = _sc_gather_fused(idx3, par3, table2)
    loc_embedded = x.transpose(2, 4, 0, 1, 3).reshape(_NB, _NS, _DIM)
    user_t_out, time_out, smooth_out = _tc_small(
        user_table.T, time_table, jnp.asarray(_KMAT)
    )
    return (loc_embedded, time_out, smooth_out, user_t_out.T)


# speedup vs baseline: 2.0133x; 1.0543x over previous
import functools

import numpy as np
import jax
import jax.numpy as jnp
from jax import lax
from jax.experimental import pallas as pl
from jax.experimental.pallas import tpu as pltpu
from jax.experimental.pallas import tpu_sc as plsc

_DIM = 64
_NB = 4096
_NS = 200
_NW = 32
_BT = _NB // 128
_HALF = 524_288
_BANDWIDTH = 2.0


def _gaussian_kernel_const() -> np.ndarray:
    t = np.arange(24, dtype=np.float32)
    ad = np.abs(t[None, :] - t[:, None])
    dist = np.minimum(ad, 24.0 - ad)
    return np.exp(-0.5 * (dist / _BANDWIDTH) ** 2).astype(np.float32)


_KMAT = _gaussian_kernel_const()


def _tc_pack(table_t):
    w = 8192
    g = _HALF // w
    jmax = 1_000_000 // w

    def body(a_ref, b_ref, eye_ref, out_ref):
        eye = eye_ref[...]
        dn = (((0,), (0,)), ((), ()))
        out_ref[:, 0:_DIM] = lax.dot_general(
            a_ref[...], eye, dn, preferred_element_type=jnp.float32)
        out_ref[:, _DIM:128] = lax.dot_general(
            b_ref[...], eye, dn, preferred_element_type=jnp.float32)

    return pl.pallas_call(
        body,
        grid=(g,),
        in_specs=[
            pl.BlockSpec((_DIM, w), lambda i: (0, i)),
            pl.BlockSpec((_DIM, w), lambda i: (0, jnp.minimum(i + g, jmax))),
            pl.BlockSpec((_DIM, _DIM), lambda i: (0, 0)),
        ],
        out_specs=pl.BlockSpec((w, 128), lambda i: (i, 0)),
        out_shape=jax.ShapeDtypeStruct((_HALF, 128), jnp.float32),
    )(table_t, table_t, jnp.eye(_DIM, dtype=jnp.float32))


def _sc_gather_fused(idx3, par3, table2):
    mesh = plsc.VectorSubcoreMesh(core_axis_name="c", subcore_axis_name="s")

    @functools.partial(
        pl.kernel,
        mesh=mesh,
        out_type=jax.ShapeDtypeStruct((_NS, 8, _BT, 8, 128), jnp.float32),
        scratch_types=[
            pltpu.VMEM((_NS, 128), jnp.int32),
            pltpu.VMEM((_NS, 128), jnp.int32),
            pltpu.VMEM((2, 128, 128), jnp.float32),
            pltpu.VMEM((2, 8, 8, 128), jnp.float32),
            pltpu.SemaphoreType.DMA,
            pltpu.SemaphoreType.DMA,
            pltpu.SemaphoreType.DMA,
            pltpu.SemaphoreType.DMA,
        ],
        compiler_params=pltpu.CompilerParams(
            use_tc_tiling_on_sc=False, needs_layout_passes=False
        ),
    )
    def gather_kernel(idx_hbm, par_hbm, table_hbm, out_hbm,
                      idxv, parv, rowbuf, tbuf, gsem0, gsem1, wsem0, wsem1):
        w = lax.axis_index("s") * 2 + lax.axis_index("c")
        gsems = (gsem0, gsem1)
        wsems = (wsem0, wsem1)
        pltpu.sync_copy(idx_hbm.at[:, w], idxv)
        pltpu.sync_copy(par_hbm.at[:, w], parv)

        lvec = lax.iota(jnp.int32, 16)
        doffs = [(lvec + i) & 15 for i in range(16)]

        def gather_desc(s, b):
            return pltpu.make_async_copy(
                table_hbm.at[idxv.at[s]], rowbuf.at[b], gsems[b])

        def write_desc(s, b):
            return pltpu.make_async_copy(
                tbuf.at[b], out_hbm.at[s, :, w], wsems[b])

        def transpose(s, b):
            def blk(kb, carry):
                rows = kb * 16 + lvec
                par16 = parv[s, pl.ds(kb * 16, 16)]
                for kd in range(4):
                    base = kd * 16 + par16
                    for step in range(16):
                        doff = doffs[step]
                        v = plsc.load_gather(
                            rowbuf.at[b], [rows, base + doff])
                        plsc.store_scatter(
                            tbuf.at[b],
                            [2 * kd + (doff >> 3), doff & 7, rows], v)
                return carry

            lax.fori_loop(0, 8, blk, 0)

        gather_desc(0, 0).start()
        gather_desc(1, 1).start()

        def body(g, carry):
            for b in (0, 1):
                s = 2 * g + b
                gather_desc(s, b).wait()

                @pl.when(g >= 1)
                def _drain():
                    write_desc(s - 2, b).wait()

                transpose(s, b)

                @pl.when(g < _NS // 2 - 1)
                def _next():
                    gather_desc(s + 2, b).start()

                write_desc(s, b).start()

            return carry

        lax.fori_loop(0, _NS // 2, body, 0)
        write_desc(_NS - 2, 0).wait()
        write_desc(_NS - 1, 1).wait()

    return gather_kernel(idx3, par3, table2)


def _tc_small(user_t, time_table, kmat):
    nu = user_t.shape[1]
    blk = 8192
    grid = (nu + blk - 1) // blk

    def body(user_ref, time_ref, kmat_ref, user_out, time_out, smooth_out):
        i = pl.program_id(0)
        user_out[...] = user_ref[...]

        @pl.when(i == 0)
        def _():
            time_out[...] = time_ref[...]
            smooth_out[...] = jnp.dot(
                kmat_ref[...], time_ref[...], preferred_element_type=jnp.float32
            )

    return pl.pallas_call(
        body,
        grid=(grid,),
        in_specs=[
            pl.BlockSpec((_DIM, blk), lambda i: (0, i)),
            pl.BlockSpec((24, _DIM), lambda i: (0, 0)),
            pl.BlockSpec((24, 24), lambda i: (0, 0)),
        ],
        out_specs=[
            pl.BlockSpec((_DIM, blk), lambda i: (0, i)),
            pl.BlockSpec((24, _DIM), lambda i: (0, 0)),
            pl.BlockSpec((24, _DIM), lambda i: (0, 0)),
        ],
        out_shape=[
            jax.ShapeDtypeStruct((_DIM, nu), jnp.float32),
            jax.ShapeDtypeStruct((24, _DIM), jnp.float32),
            jax.ShapeDtypeStruct((24, _DIM), jnp.float32),
        ],
    )(user_t, time_table, kmat)


def kernel(location_x, loc_table, user_table, time_table):
    locs = jnp.transpose(location_x.astype(jnp.int32)).reshape(_NS, _BT, 128)
    hi = locs >= _HALF
    idx3 = jnp.where(hi, locs - _HALF, locs)
    par3 = jnp.where(hi, _DIM, 0).astype(jnp.int32)
    table2 = _tc_pack(loc_table.T)
    x = _sc_gather_fused(idx3, par3, table2)
    loc_embedded = x.transpose(2, 4, 0, 1, 3).reshape(_NB, _NS, _DIM)
    user_t_out, time_out, smooth_out = _tc_small(
        user_table.T, time_table, jnp.asarray(_KMAT)
    )
    return (loc_embedded, time_out, smooth_out, user_t_out.T)
